# Initial kernel scaffold; baseline (speedup 1.0000x reference)
import jax
import jax.numpy as jnp
from jax.experimental import pallas as pl


def kernel(features, edge_index, W, a, b):
    raise NotImplementedError("write your pallas kernel here")


# speedup vs baseline: 7.5327x; 7.5327x over previous
import functools

import jax
import jax.numpy as jnp
from jax import lax
from jax.experimental import pallas as pl
from jax.experimental.pallas import tpu as pltpu
from jax.experimental.pallas import tpu_sc as plsc

N = 10000
D = 128
E = 320000
ALPHA = 0.2

NC = 2
NS = 16
NW = NC * NS
EPW = E // NW
C = 200
NCH = EPW // C
OCHK = 200
NOCHK = N // OCHK

ROW_BLK = 1000


def _prep_body(x_ref, w_ref, b_ref, a1_ref, a2_ref, feat_ref, s1_ref, s2_ref):
    feat = jnp.dot(x_ref[...], w_ref[...], preferred_element_type=jnp.float32)
    feat = feat + b_ref[...]
    feat_ref[...] = feat
    s1_ref[...] = jnp.dot(feat, a1_ref[...], preferred_element_type=jnp.float32)
    s2_ref[...] = jnp.dot(feat, a2_ref[...], preferred_element_type=jnp.float32)


def _prep(x, w, b2d, a1p, a2p):
    grid = N // ROW_BLK
    return pl.pallas_call(
        _prep_body,
        grid=(grid,),
        in_specs=[
            pl.BlockSpec((ROW_BLK, D), lambda i: (i, 0)),
            pl.BlockSpec((D, D), lambda i: (0, 0)),
            pl.BlockSpec((1, D), lambda i: (0, 0)),
            pl.BlockSpec((D, 16), lambda i: (0, 0)),
            pl.BlockSpec((D, 16), lambda i: (0, 0)),
        ],
        out_specs=[
            pl.BlockSpec((ROW_BLK, D), lambda i: (i, 0)),
            pl.BlockSpec((ROW_BLK, 16), lambda i: (i, 0)),
            pl.BlockSpec((ROW_BLK, 16), lambda i: (i, 0)),
        ],
        out_shape=[
            jax.ShapeDtypeStruct((N, D), jnp.float32),
            jax.ShapeDtypeStruct((N, 16), jnp.float32),
            jax.ShapeDtypeStruct((N, 16), jnp.float32),
        ],
    )(x, w, b2d, a1p, a2p)


def _edge_body(src_hbm, dst_hbm, feat_hbm, s1_hbm, s2_hbm, out_hbm,
               acc, src_v, dst_v, rows_v, s1_v, s2_v, sem0, sem1, sem2):
    cid = lax.axis_index("c")
    sid = lax.axis_index("s")
    wid = cid * NS + sid

    zero16 = jnp.zeros((16,), jnp.float32)

    def zbody(e, carry):
        for dd in range(D // 16):
            rows_v[e, pl.ds(dd * 16, 16)] = zero16
        return carry
    lax.fori_loop(0, C, zbody, 0)

    for k in range((NOCHK + NS - 1) // NS):
        ch = k * NS + sid

        @pl.when(ch < NOCHK)
        def _():
            pltpu.sync_copy(rows_v.at[pl.ds(0, OCHK)],
                            acc.at[pl.ds(ch * OCHK, OCHK)])
    plsc.subcore_barrier()

    def chunk_body(ch, carry):
        base = pl.multiple_of(wid * EPW + ch * C, 8)
        pltpu.sync_copy(src_hbm.at[pl.ds(base, C)], src_v)
        pltpu.sync_copy(dst_hbm.at[pl.ds(base, C)], dst_v)
        cp0 = pltpu.async_copy(feat_hbm.at[dst_v], rows_v, sem0)
        cp1 = pltpu.async_copy(s1_hbm.at[src_v], s1_v, sem1)
        cp2 = pltpu.async_copy(s2_hbm.at[dst_v], s2_v, sem2)
        cp1.wait()
        cp2.wait()
        cp0.wait()

        def edge_body(e, c2):
            t16 = s1_v[e, pl.ds(0, 16)] + s2_v[e, pl.ds(0, 16)]
            v16 = jnp.where(t16 > 0.0, t16, t16 * ALPHA)
            v = v16[0]
            for dd in range(D // 16):
                sl = pl.ds(dd * 16, 16)
                rows_v[e, sl] = rows_v[e, sl] * v
            return c2
        lax.fori_loop(0, C, edge_body, 0)

        pltpu.sync_copy(rows_v, acc.at[src_v], add=True)
        return carry

    lax.fori_loop(0, NCH, chunk_body, 0)
    plsc.subcore_barrier()

    for k in range((NOCHK + NS - 1) // NS):
        ch = k * NS + sid

        @pl.when(ch < NOCHK)
        def _():
            pltpu.sync_copy(acc.at[pl.ds(ch * OCHK, OCHK)],
                            rows_v.at[pl.ds(0, OCHK)])
            pltpu.sync_copy(rows_v.at[pl.ds(0, OCHK)],
                            out_hbm.at[cid, pl.ds(ch * OCHK, OCHK)])


_edge = functools.partial(
    pl.kernel,
    out_type=jax.ShapeDtypeStruct((NC, N, D), jnp.float32),
    mesh=plsc.VectorSubcoreMesh(core_axis_name="c", subcore_axis_name="s"),
    compiler_params=pltpu.CompilerParams(use_tc_tiling_on_sc=False),
    scratch_types=[
        pltpu.VMEM_SHARED((N, D), jnp.float32),
        pltpu.VMEM((C,), jnp.int32),
        pltpu.VMEM((C,), jnp.int32),
        pltpu.VMEM((C, D), jnp.float32),
        pltpu.VMEM((C, 16), jnp.float32),
        pltpu.VMEM((C, 16), jnp.float32),
        pltpu.SemaphoreType.DMA,
        pltpu.SemaphoreType.DMA,
        pltpu.SemaphoreType.DMA,
    ],
)(_edge_body)


def _combine_body(p_ref, o_ref):
    o_ref[...] = p_ref[0] + p_ref[1]


def _combine(partial):
    grid = N // ROW_BLK
    return pl.pallas_call(
        _combine_body,
        grid=(grid,),
        in_specs=[pl.BlockSpec((NC, ROW_BLK, D), lambda i: (0, i, 0))],
        out_specs=pl.BlockSpec((ROW_BLK, D), lambda i: (i, 0)),
        out_shape=jax.ShapeDtypeStruct((N, D), jnp.float32),
    )(partial)


def kernel(features, edge_index, W, a, b):
    src = edge_index[0].astype(jnp.int32)
    dst = edge_index[1].astype(jnp.int32)
    b2d = b.reshape(1, D)
    a1p = jnp.zeros((D, 16), jnp.float32).at[:, 0].set(a[:D, 0])
    a2p = jnp.zeros((D, 16), jnp.float32).at[:, 0].set(a[D:, 0])

    feat, s1t, s2t = _prep(features, W, b2d, a1p, a2p)
    partial = _edge(src, dst, feat, s1t, s2t)
    return _combine(partial)


# speedup vs baseline: 8.4546x; 1.1224x over previous
import functools

import jax
import jax.numpy as jnp
from jax import lax
from jax.experimental import pallas as pl
from jax.experimental.pallas import tpu as pltpu
from jax.experimental.pallas import tpu_sc as plsc

N = 10000
D = 128
E = 320000
ALPHA = 0.2

NC = 2
NS = 16
NW = NC * NS
EPW = E // NW
C = 80
NCH = EPW // C
OCHK = 80
NOCHK = N // OCHK

ROW_BLK = 1000


def _prep_body(x_ref, w_ref, b_ref, a1_ref, a2_ref, feat_ref, s1_ref, s2_ref):
    feat = jnp.dot(x_ref[...], w_ref[...], preferred_element_type=jnp.float32)
    feat = feat + b_ref[...]
    feat_ref[...] = feat
    s1_ref[...] = jnp.dot(feat, a1_ref[...], preferred_element_type=jnp.float32)
    s2_ref[...] = jnp.dot(feat, a2_ref[...], preferred_element_type=jnp.float32)


def _prep(x, w, b2d, a1p, a2p):
    grid = N // ROW_BLK
    return pl.pallas_call(
        _prep_body,
        grid=(grid,),
        in_specs=[
            pl.BlockSpec((ROW_BLK, D), lambda i: (i, 0)),
            pl.BlockSpec((D, D), lambda i: (0, 0)),
            pl.BlockSpec((1, D), lambda i: (0, 0)),
            pl.BlockSpec((D, 16), lambda i: (0, 0)),
            pl.BlockSpec((D, 16), lambda i: (0, 0)),
        ],
        out_specs=[
            pl.BlockSpec((ROW_BLK, D), lambda i: (i, 0)),
            pl.BlockSpec((ROW_BLK, 16), lambda i: (i, 0)),
            pl.BlockSpec((ROW_BLK, 16), lambda i: (i, 0)),
        ],
        out_shape=[
            jax.ShapeDtypeStruct((N, D), jnp.float32),
            jax.ShapeDtypeStruct((N, 16), jnp.float32),
            jax.ShapeDtypeStruct((N, 16), jnp.float32),
        ],
    )(x, w, b2d, a1p, a2p)


def _edge_body(src_hbm, dst_hbm, feat_hbm, s1_hbm, s2_hbm, out_hbm,
               acc, src_v, dst_v, rows_v, s1_v, s2_v, sem_g, sem_s):
    cid = lax.axis_index("c")
    sid = lax.axis_index("s")
    wid = cid * NS + sid

    zero16 = jnp.zeros((16,), jnp.float32)

    def zbody(e, carry):
        for dd in range(D // 16):
            rows_v[0][e, pl.ds(dd * 16, 16)] = zero16
        return carry
    lax.fori_loop(0, C, zbody, 0)

    for k in range((NOCHK + NS - 1) // NS):
        ch = k * NS + sid

        @pl.when(ch < NOCHK)
        def _():
            pltpu.sync_copy(rows_v[0].at[pl.ds(0, OCHK)],
                            acc.at[pl.ds(ch * OCHK, OCHK)])

    def idx_load(n, p):
        base = pl.multiple_of(wid * EPW + n * C, 8)
        pltpu.sync_copy(src_hbm.at[pl.ds(base, C)], src_v[p])
        pltpu.sync_copy(dst_hbm.at[pl.ds(base, C)], dst_v[p])

    def issue_gathers(p):
        pltpu.async_copy(feat_hbm.at[dst_v[p]], rows_v[p], sem_g[p])
        pltpu.async_copy(s1_hbm.at[src_v[p]], s1_v[p], sem_g[p])
        pltpu.async_copy(s2_hbm.at[dst_v[p]], s2_v[p], sem_g[p])

    def wait_gathers(p):
        pltpu.make_async_copy(feat_hbm.at[dst_v[p]], rows_v[p], sem_g[p]).wait()
        pltpu.make_async_copy(s1_hbm.at[src_v[p]], s1_v[p], sem_g[p]).wait()
        pltpu.make_async_copy(s2_hbm.at[dst_v[p]], s2_v[p], sem_g[p]).wait()

    def issue_scatter(p):
        pltpu.async_copy(rows_v[p], acc.at[src_v[p]], sem_s[p], add=True)

    def wait_scatter(p):
        pltpu.make_async_copy(rows_v[p], acc.at[src_v[p]], sem_s[p]).wait()

    def compute(p):
        def grp(e, c2):
            t16 = s1_v[p][e, pl.ds(0, 16)] + s2_v[p][e, pl.ds(0, 16)]
            v16 = jnp.where(t16 > 0.0, t16, t16 * ALPHA)
            for dd in range(D // 16):
                sl = pl.ds(dd * 16, 16)
                rows_v[p][e, sl] = rows_v[p][e, sl] * v16
            return c2
        lax.fori_loop(0, C, grp, 0)

    idx_load(0, 0)
    issue_gathers(0)
    plsc.subcore_barrier()

    def step(n, p, g):
        wait_gathers(p)
        if g is not None:
            @pl.when(g > 0)
            def _():
                wait_scatter(1 - p)
        else:
            wait_scatter(1 - p)
        idx_load(n + 1, 1 - p)
        issue_gathers(1 - p)
        compute(p)
        issue_scatter(p)

    def pair(g, carry):
        step(2 * g, 0, g)
        step(2 * g + 1, 1, None)
        return carry
    lax.fori_loop(0, (NCH - 1) // 2, pair, 0)

    wait_gathers(0)
    wait_scatter(1)
    compute(0)
    issue_scatter(0)
    wait_scatter(0)

    plsc.subcore_barrier()

    for k in range((NOCHK + NS - 1) // NS):
        ch = k * NS + sid

        @pl.when(ch < NOCHK)
        def _():
            pltpu.sync_copy(acc.at[pl.ds(ch * OCHK, OCHK)], rows_v[0])
            pltpu.sync_copy(rows_v[0],
                            out_hbm.at[cid, pl.ds(ch * OCHK, OCHK)])


_edge = functools.partial(
    pl.kernel,
    out_type=jax.ShapeDtypeStruct((NC, N, D), jnp.float32),
    mesh=plsc.VectorSubcoreMesh(core_axis_name="c", subcore_axis_name="s"),
    compiler_params=pltpu.CompilerParams(use_tc_tiling_on_sc=False),
    scratch_types=[
        pltpu.VMEM_SHARED((N, D), jnp.float32),
        [pltpu.VMEM((C,), jnp.int32)] * 2,
        [pltpu.VMEM((C,), jnp.int32)] * 2,
        [pltpu.VMEM((C, D), jnp.float32)] * 2,
        [pltpu.VMEM((C, 16), jnp.float32)] * 2,
        [pltpu.VMEM((C, 16), jnp.float32)] * 2,
        [pltpu.SemaphoreType.DMA] * 2,
        [pltpu.SemaphoreType.DMA] * 2,
    ],
)(_edge_body)


def _combine_body(p_ref, o_ref):
    o_ref[...] = p_ref[0] + p_ref[1]


def _combine(partial):
    grid = N // ROW_BLK
    return pl.pallas_call(
        _combine_body,
        grid=(grid,),
        in_specs=[pl.BlockSpec((NC, ROW_BLK, D), lambda i: (0, i, 0))],
        out_specs=pl.BlockSpec((ROW_BLK, D), lambda i: (i, 0)),
        out_shape=jax.ShapeDtypeStruct((N, D), jnp.float32),
    )(partial)


def kernel(features, edge_index, W, a, b):
    src = edge_index[0].astype(jnp.int32)
    dst = edge_index[1].astype(jnp.int32)
    b2d = b.reshape(1, D)
    a1p = jnp.tile(a[:D], (1, 16))
    a2p = jnp.tile(a[D:], (1, 16))

    feat, s1t, s2t = _prep(features, W, b2d, a1p, a2p)
    partial = _edge(src, dst, feat, s1t, s2t)
    return _combine(partial)


# speedup vs baseline: 12.0155x; 1.4212x over previous
import functools

import jax
import jax.numpy as jnp
from jax import lax
from jax.experimental import pallas as pl
from jax.experimental.pallas import tpu as pltpu
from jax.experimental.pallas import tpu_sc as plsc

N = 10000
D = 128
E = 320000
ALPHA = 0.2

NC = 2
NS = 16
NW = NC * NS
EPW = E // NW
C = 80
NCH = EPW // C
OCHK = 80
NOCHK = N // OCHK

ROW_BLK = 1000


def _prep_body(x_ref, w_ref, b_ref, a1_ref, a2_ref, feat_ref, s1_ref, s2_ref):
    feat = jnp.dot(x_ref[...], w_ref[...], preferred_element_type=jnp.float32)
    feat = feat + b_ref[...]
    feat_ref[...] = feat
    s1_ref[...] = jnp.dot(feat, a1_ref[...], preferred_element_type=jnp.float32)
    s2_ref[...] = jnp.dot(feat, a2_ref[...], preferred_element_type=jnp.float32)


def _prep(x, w, b2d, a1p, a2p):
    grid = N // ROW_BLK
    return pl.pallas_call(
        _prep_body,
        grid=(grid,),
        in_specs=[
            pl.BlockSpec((ROW_BLK, D), lambda i: (i, 0)),
            pl.BlockSpec((D, D), lambda i: (0, 0)),
            pl.BlockSpec((1, D), lambda i: (0, 0)),
            pl.BlockSpec((D, 16), lambda i: (0, 0)),
            pl.BlockSpec((D, 16), lambda i: (0, 0)),
        ],
        out_specs=[
            pl.BlockSpec((ROW_BLK, D), lambda i: (i, 0)),
            pl.BlockSpec((ROW_BLK, 16), lambda i: (i, 0)),
            pl.BlockSpec((ROW_BLK, 16), lambda i: (i, 0)),
        ],
        out_shape=[
            jax.ShapeDtypeStruct((N, D), jnp.float32),
            jax.ShapeDtypeStruct((N, 16), jnp.float32),
            jax.ShapeDtypeStruct((N, 16), jnp.float32),
        ],
    )(x, w, b2d, a1p, a2p)


def _edge_body(src_hbm, dst_hbm, feat_hbm, s1_hbm, s2_hbm, out_hbm,
               acc, srcs_v, dsts_v, rows_v, s1_v, s2_v, sem_g, sem_s):
    cid = lax.axis_index("c")
    sid = lax.axis_index("s")
    wid = cid * NS + sid

    zero16 = jnp.zeros((16,), jnp.float32)

    def zbody(e, carry):
        for dd in range(D // 16):
            rows_v[0][e, pl.ds(dd * 16, 16)] = zero16
        return carry
    lax.fori_loop(0, C, zbody, 0)

    for k in range((NOCHK + NS - 1) // NS):
        ch = k * NS + sid

        @pl.when(ch < NOCHK)
        def _():
            pltpu.sync_copy(rows_v[0].at[pl.ds(0, OCHK)],
                            acc.at[pl.ds(ch * OCHK, OCHK)])

    pltpu.sync_copy(src_hbm.at[pl.ds(wid * NCH, NCH)], srcs_v)
    pltpu.sync_copy(dst_hbm.at[pl.ds(wid * NCH, NCH)], dsts_v)

    def issue_gathers(n, p):
        pltpu.async_copy(feat_hbm.at[dsts_v.at[n]], rows_v[p], sem_g[p])
        pltpu.async_copy(s1_hbm.at[srcs_v.at[n]], s1_v[p], sem_g[p])
        pltpu.async_copy(s2_hbm.at[dsts_v.at[n]], s2_v[p], sem_g[p])

    def wait_gathers(n, p):
        pltpu.make_async_copy(feat_hbm.at[dsts_v.at[n]], rows_v[p], sem_g[p]).wait()
        pltpu.make_async_copy(s1_hbm.at[srcs_v.at[n]], s1_v[p], sem_g[p]).wait()
        pltpu.make_async_copy(s2_hbm.at[dsts_v.at[n]], s2_v[p], sem_g[p]).wait()

    def issue_scatter(n, p):
        pltpu.async_copy(rows_v[p], acc.at[srcs_v.at[n]], sem_s[p], add=True)

    def wait_scatter(n, p):
        pltpu.make_async_copy(rows_v[p], acc.at[srcs_v.at[n]], sem_s[p]).wait()

    def compute(p):
        def grp(e, c2):
            t16 = s1_v[p][e, pl.ds(0, 16)] + s2_v[p][e, pl.ds(0, 16)]
            v16 = jnp.where(t16 > 0.0, t16, t16 * ALPHA)
            for dd in range(D // 16):
                sl = pl.ds(dd * 16, 16)
                rows_v[p][e, sl] = rows_v[p][e, sl] * v16
            return c2
        lax.fori_loop(0, C, grp, 0)

    issue_gathers(0, 0)
    plsc.subcore_barrier()

    def step(n, p, g):
        wait_gathers(n, p)
        if g is not None:
            @pl.when(g > 0)
            def _():
                wait_scatter(n - 1, 1 - p)
        else:
            wait_scatter(n - 1, 1 - p)
        issue_gathers(n + 1, 1 - p)
        compute(p)
        issue_scatter(n, p)

    def pair(g, carry):
        step(2 * g, 0, g)
        step(2 * g + 1, 1, None)
        return carry
    lax.fori_loop(0, (NCH - 1) // 2, pair, 0)

    wait_gathers(NCH - 1, 0)
    wait_scatter(NCH - 2, 1)
    compute(0)
    issue_scatter(NCH - 1, 0)
    wait_scatter(NCH - 1, 0)

    plsc.subcore_barrier()

    for k in range((NOCHK + NS - 1) // NS):
        ch = k * NS + sid

        @pl.when(ch < NOCHK)
        def _():
            pltpu.sync_copy(acc.at[pl.ds(ch * OCHK, OCHK)], rows_v[0])
            pltpu.sync_copy(rows_v[0],
                            out_hbm.at[cid, pl.ds(ch * OCHK, OCHK)])


_edge = functools.partial(
    pl.kernel,
    out_type=jax.ShapeDtypeStruct((NC, N, D), jnp.float32),
    mesh=plsc.VectorSubcoreMesh(core_axis_name="c", subcore_axis_name="s"),
    compiler_params=pltpu.CompilerParams(use_tc_tiling_on_sc=False),
    scratch_types=[
        pltpu.VMEM_SHARED((N, D), jnp.float32),
        pltpu.VMEM((NCH, C), jnp.int32),
        pltpu.VMEM((NCH, C), jnp.int32),
        [pltpu.VMEM((C, D), jnp.float32)] * 2,
        [pltpu.VMEM((C, 16), jnp.float32)] * 2,
        [pltpu.VMEM((C, 16), jnp.float32)] * 2,
        [pltpu.SemaphoreType.DMA] * 2,
        [pltpu.SemaphoreType.DMA] * 2,
    ],
)(_edge_body)


def _combine_body(p_ref, o_ref):
    o_ref[...] = p_ref[0] + p_ref[1]


def _combine(partial):
    grid = N // ROW_BLK
    return pl.pallas_call(
        _combine_body,
        grid=(grid,),
        in_specs=[pl.BlockSpec((NC, ROW_BLK, D), lambda i: (0, i, 0))],
        out_specs=pl.BlockSpec((ROW_BLK, D), lambda i: (i, 0)),
        out_shape=jax.ShapeDtypeStruct((N, D), jnp.float32),
    )(partial)


def kernel(features, edge_index, W, a, b):
    src = edge_index[0].astype(jnp.int32).reshape(NW * NCH, C)
    dst = edge_index[1].astype(jnp.int32).reshape(NW * NCH, C)
    b2d = b.reshape(1, D)
    a1p = jnp.tile(a[:D], (1, 16))
    a2p = jnp.tile(a[D:], (1, 16))

    feat, s1t, s2t = _prep(features, W, b2d, a1p, a2p)
    partial = _edge(src, dst, feat, s1t, s2t)
    return _combine(partial)
